# Initial kernel scaffold; baseline (speedup 1.0000x reference)
import jax
import jax.numpy as jnp
from jax.experimental import pallas as pl


def kernel(hidden_states, gate_w, expert_gate_w, expert_up_w, expert_down_w, shared_gate_w, shared_up_w, shared_down_w, shared_expert_gate_w):
    raise NotImplementedError("write your pallas kernel here")


# speedup vs baseline: 1.1309x; 1.1309x over previous
import functools

import jax
import jax.numpy as jnp
from jax.experimental import pallas as pl
from jax.experimental.pallas import tpu as pltpu

B = 1
S = 2048
D = 2048
E = 8
K = 2
F = 1408
SF = 5632

T = B * S
N = T * K

BM = 256
NUM_TILES = N // BM
V = NUM_TILES + E - 1
BF = 704
NF = F // BF

BMS = 512
BFS = 704
NFS = SF // BFS

BMR = 512


def _gmm_kernel(tile_ids, group_ids, row_start, row_end,
                x_ref, w_ref, wg_ref, wu_ref, wd_ref, out_ref):
    i = pl.program_id(0)
    f = pl.program_id(1)
    tile = tile_ids[i]
    rows = tile * BM + jax.lax.broadcasted_iota(jnp.int32, (BM, 1), 0)
    mask = (rows >= row_start[i]) & (rows < row_end[i])
    w = jnp.where(mask, w_ref[...], 0.0)
    x = x_ref[...]
    g = jax.lax.dot_general(x, wg_ref[0], (((1,), (1,)), ((), ())),
                            preferred_element_type=jnp.float32)
    u = jax.lax.dot_general(x, wu_ref[0], (((1,), (1,)), ((), ())),
                            preferred_element_type=jnp.float32)
    h = (g * jax.nn.sigmoid(g)) * u * w
    hb = h.astype(jnp.bfloat16)
    o = jax.lax.dot_general(hb, wd_ref[0], (((1,), (0,)), ((), ())),
                            preferred_element_type=jnp.float32)
    prev_tile = tile_ids[jnp.maximum(i - 1, 0)]
    first = (f == 0) & ((i == 0) | (tile != prev_tile))

    @pl.when(first)
    def _():
        out_ref[...] = o

    @pl.when(jnp.logical_not(first))
    def _():
        out_ref[...] += o


def _shared_kernel(x_ref, sgw_ref, wg_ref, wu_ref, wd_ref, out_ref):
    f = pl.program_id(1)
    x = x_ref[...]
    sgate = jax.nn.sigmoid(jnp.sum(
        x.astype(jnp.float32) * sgw_ref[...].astype(jnp.float32),
        axis=1, keepdims=True))
    g = jax.lax.dot_general(x, wg_ref[...], (((1,), (1,)), ((), ())),
                            preferred_element_type=jnp.float32)
    u = jax.lax.dot_general(x, wu_ref[...], (((1,), (1,)), ((), ())),
                            preferred_element_type=jnp.float32)
    h = (g * jax.nn.sigmoid(g)) * u * sgate
    hb = h.astype(jnp.bfloat16)
    o = jax.lax.dot_general(hb, wd_ref[...], (((1,), (0,)), ((), ())),
                            preferred_element_type=jnp.float32)

    @pl.when(f == 0)
    def _():
        out_ref[...] = o

    @pl.when(f != 0)
    def _():
        out_ref[...] += o


def kernel(hidden_states, gate_w, expert_gate_w, expert_up_w, expert_down_w,
           shared_gate_w, shared_up_w, shared_down_w, shared_expert_gate_w):
    x = hidden_states.reshape(T, D)

    logits = x @ gate_w.T
    probs = jax.nn.softmax(logits.astype(jnp.float32), axis=1)
    wts, ids = jax.lax.top_k(probs, K)

    flat_e = ids.reshape(-1)
    order = jnp.argsort(flat_e, stable=True).astype(jnp.int32)
    token_sorted = order // K
    w_sorted = wts.reshape(-1)[order][:, None]

    sizes = jnp.bincount(flat_e, length=E).astype(jnp.int32)
    ends = jnp.cumsum(sizes)
    starts = ends - sizes
    t_first = starts // BM
    t_last = jnp.where(sizes > 0, (ends - 1) // BM, 0)
    visits = jnp.where(sizes > 0, t_last - t_first + 1, 0)
    vend = jnp.cumsum(visits)
    vstart = vend - visits
    total = vend[E - 1]
    slots = jnp.arange(V, dtype=jnp.int32)
    e_of = jnp.searchsorted(vend, slots, side='right').astype(jnp.int32)
    valid = slots < total
    e_cl = jnp.minimum(e_of, E - 1)
    tiles = t_first[e_cl] + (slots - vstart[e_cl])
    tile_ids = jnp.where(valid, tiles, NUM_TILES - 1).astype(jnp.int32)
    group_ids = e_cl
    row_start = jnp.where(valid, starts[e_cl], 1).astype(jnp.int32)
    row_end = jnp.where(valid, ends[e_cl], 0).astype(jnp.int32)

    x_bf = x.astype(jnp.bfloat16)
    x_sorted = jnp.take(x_bf, token_sorted, axis=0)
    egw = expert_gate_w.astype(jnp.bfloat16)
    euw = expert_up_w.astype(jnp.bfloat16)
    edw = expert_down_w.astype(jnp.bfloat16).transpose(0, 2, 1)

    out_sorted = pl.pallas_call(
        _gmm_kernel,
        grid_spec=pltpu.PrefetchScalarGridSpec(
            num_scalar_prefetch=4,
            grid=(V, NF),
            in_specs=[
                pl.BlockSpec((BM, D), lambda i, f, ti, gi, rs, re: (ti[i], 0)),
                pl.BlockSpec((BM, 1), lambda i, f, ti, gi, rs, re: (ti[i], 0)),
                pl.BlockSpec((1, BF, D), lambda i, f, ti, gi, rs, re: (gi[i], f, 0)),
                pl.BlockSpec((1, BF, D), lambda i, f, ti, gi, rs, re: (gi[i], f, 0)),
                pl.BlockSpec((1, BF, D), lambda i, f, ti, gi, rs, re: (gi[i], f, 0)),
            ],
            out_specs=pl.BlockSpec((BM, D), lambda i, f, ti, gi, rs, re: (ti[i], 0)),
        ),
        out_shape=jax.ShapeDtypeStruct((N, D), jnp.float32),
        compiler_params=pltpu.CompilerParams(
            dimension_semantics=("arbitrary", "arbitrary")),
    )(tile_ids, group_ids, row_start, row_end,
      x_sorted, w_sorted, egw, euw, edw)

    sgw = shared_gate_w.astype(jnp.bfloat16)
    suw = shared_up_w.astype(jnp.bfloat16)
    sdw = shared_down_w.astype(jnp.bfloat16).T
    segw = shared_expert_gate_w.astype(jnp.bfloat16)
    shared_out = pl.pallas_call(
        _shared_kernel,
        grid=(T // BMS, NFS),
        in_specs=[
            pl.BlockSpec((BMS, D), lambda m, f: (m, 0)),
            pl.BlockSpec((1, D), lambda m, f: (0, 0)),
            pl.BlockSpec((BFS, D), lambda m, f: (f, 0)),
            pl.BlockSpec((BFS, D), lambda m, f: (f, 0)),
            pl.BlockSpec((BFS, D), lambda m, f: (f, 0)),
        ],
        out_specs=pl.BlockSpec((BMS, D), lambda m, f: (m, 0)),
        out_shape=jax.ShapeDtypeStruct((T, D), jnp.float32),
        compiler_params=pltpu.CompilerParams(
            dimension_semantics=("arbitrary", "arbitrary")),
    )(x_bf, segw, sgw, suw, sdw)

    inv = jnp.argsort(order).astype(jnp.int32)
    pos = inv.reshape(T, K)
    final = (jnp.take(out_sorted, pos[:, 0], axis=0)
             + jnp.take(out_sorted, pos[:, 1], axis=0)
             + shared_out)
    return final.reshape(B, S, D), logits
